# Initial kernel scaffold; baseline (speedup 1.0000x reference)
import jax
import jax.numpy as jnp
from jax.experimental import pallas as pl


def kernel(values, coords, spacing, shift):
    raise NotImplementedError("write your pallas kernel here")


# speedup vs baseline: 1.5581x; 1.5581x over previous
import functools

import jax
import jax.numpy as jnp
from jax import lax
from jax.experimental import pallas as pl
from jax.experimental.pallas import tpu as pltpu
from jax.experimental.pallas import tpu_sc as plsc

N = 4096
C = 256
NV = 3
BQ = 256
NB = N // BQ
B_OUT = 4 * N

NC = 2
NS = 16
NW = NC * NS
BPW = B_OUT // NW
CH = 128
NCH = BPW // CH


def _argmin_body(params_ref, qx_ref, qy_ref, cx_ref, cy_ref, out_ref):
    v = pl.program_id(0)
    ax = params_ref[v]
    ay = params_ref[NV + v]
    s0 = params_ref[2 * NV]
    s1 = params_ref[2 * NV + 1]
    qx = (qx_ref[...] + ax) - s0
    qy = (qy_ref[...] + ay) - s1
    dx = qx - cx_ref[...]
    dy = qy - cy_ref[...]
    dist = jnp.sqrt(dx * dx + dy * dy)
    minv = jnp.min(dist, axis=1, keepdims=True)
    iota = lax.broadcasted_iota(jnp.int32, (BQ, N), 1)
    idx = jnp.min(jnp.where(dist == minv, iota, N), axis=1)
    out_ref[0, 0, :] = idx


_argmin_call = pl.pallas_call(
    _argmin_body,
    grid=(NV, NB),
    in_specs=[
        pl.BlockSpec(memory_space=pltpu.SMEM),
        pl.BlockSpec((BQ, 1), lambda v, b: (b, 0)),
        pl.BlockSpec((BQ, 1), lambda v, b: (b, 0)),
        pl.BlockSpec((1, N), lambda v, b: (0, 0)),
        pl.BlockSpec((1, N), lambda v, b: (0, 0)),
    ],
    out_specs=pl.BlockSpec((1, 1, BQ), lambda v, b: (v * NB + b, 0, 0)),
    out_shape=jax.ShapeDtypeStruct((NV * NB, 1, BQ), jnp.int32),
)


@functools.lru_cache(maxsize=1)
def _make_sc_gather():
    mesh = plsc.VectorSubcoreMesh(core_axis_name="c", subcore_axis_name="s")

    @functools.partial(
        pl.kernel,
        mesh=mesh,
        out_type=jax.ShapeDtypeStruct((B_OUT, C), jnp.float32),
        scratch_types=[
            pltpu.VMEM((BPW,), jnp.int32),
            pltpu.VMEM((CH, C), jnp.float32),
            pltpu.SemaphoreType.DMA,
        ],
    )
    def _sc_gather(table_hbm, idx_hbm, out_hbm, idx_v, rows_v, sem):
        wid = lax.axis_index("s") * NC + lax.axis_index("c")
        base = wid * BPW
        pltpu.sync_copy(idx_hbm.at[pl.ds(base, BPW)], idx_v)
        for c in range(NCH):
            pltpu.async_copy(
                table_hbm.at[idx_v.at[pl.ds(c * CH, CH)]], rows_v, sem
            ).wait()
            pltpu.sync_copy(rows_v, out_hbm.at[pl.ds(base + c * CH, CH)])

    return _sc_gather


def kernel(values, coords, spacing, shift):
    zero = jnp.zeros((), jnp.float32)
    ax = jnp.stack([spacing[0], zero, spacing[0]])
    ay = jnp.stack([spacing[1], spacing[1], zero])
    params = jnp.concatenate([ax, ay, shift.astype(jnp.float32)])
    qx = coords[:, 0:1]
    qy = coords[:, 1:2]
    cx = coords[:, 0].reshape(1, N)
    cy = coords[:, 1].reshape(1, N)
    idx = _argmin_call(params, qx, qy, cx, cy).reshape(NV * N)
    allidx = jnp.concatenate([jnp.arange(N, dtype=jnp.int32), idx])
    return _make_sc_gather()(values, allidx)
